# baseline (speedup 1.0000x reference)
import functools

import jax
import jax.numpy as jnp
from jax import lax
from jax.experimental import pallas as pl
from jax.experimental.pallas import tpu as pltpu
from jax.experimental.pallas import tpu_sc as plsc

_H = 2
_G = 4
_K = 512
_D = 64
_BUCKETS = 65536
_HG = _H * _G
_ROUTE = _HG * _D
_TILE = 256

_DNT = (((1,), (1,)), ((), ()))


def _first_argmax(logits):
    m = jnp.max(logits, axis=-1, keepdims=True)
    kiota = jax.lax.broadcasted_iota(jnp.int32, logits.shape, 1)
    return jnp.min(jnp.where(logits == m, kiota, _K), axis=-1, keepdims=True)


def _body(aux_ref, x_ref, wr_ref, ww_ref, cbr_ref, cbw_ref,
          c0r_ref, c1r_ref, c0w_ref, c1w_ref, lr_ref, lw_ref, c2_ref):
    scale = jnp.where(aux_ref[0, 0] != 0, 1.0, 0.0).astype(jnp.float32)

    @pl.when(pl.program_id(0) == 0)
    def _init_c2():
        for s, cb_ref in ((0, cbr_ref), (1, cbw_ref)):
            for hg in range(_HG):
                tcb = cb_ref[hg].T
                c2_ref[s * _HG + hg:s * _HG + hg + 1, :] = 0.25 * jnp.sum(
                    tcb * tcb, axis=0, keepdims=True)

    x = x_ref[...]
    for s, (w_ref, cb_ref, c0_ref, c1_ref, l_ref) in enumerate((
        (wr_ref, cbr_ref, c0r_ref, c1r_ref, lr_ref),
        (ww_ref, cbw_ref, c0w_ref, c1w_ref, lw_ref),
    )):
        y = jax.lax.dot_general(x, w_ref[...], _DNT,
                                preferred_element_type=jnp.float32)
        codes = []
        for hg in range(_HG):
            yh = y[:, hg * _D:(hg + 1) * _D]
            yc2 = jax.lax.dot_general(yh, cb_ref[hg], _DNT,
                                      preferred_element_type=jnp.float32)
            y2 = jnp.sum(yh * yh, axis=1, keepdims=True)
            c2 = c2_ref[s * _HG + hg:s * _HG + hg + 1, :]
            logits = (yc2 - y2) - c2
            l_ref[:, hg * _K:(hg + 1) * _K] = logits * scale
            if hg % _G < 2:
                codes.append(_first_argmax(logits))
            else:
                codes.append(None)
        c0_ref[...] = jnp.concatenate([codes[0], codes[_G]], axis=1)
        c1_ref[...] = jnp.concatenate([codes[1], codes[_G + 1]], axis=1)


_NW = 32
_LANES = 16


def _sc_body(c0r_hbm, c1r_hbm, c0w_hbm, c1w_hbm, ir_hbm, iw_hbm, av, bv, ov):
    wid = lax.axis_index("s") * 2 + lax.axis_index("c")
    chunk = 2 * 4096 // _NW
    base = wid * chunk
    for c0_hbm, c1_hbm, o_hbm in ((c0r_hbm, c1r_hbm, ir_hbm),
                                  (c0w_hbm, c1w_hbm, iw_hbm)):
        pltpu.sync_copy(c0_hbm.at[pl.ds(base, chunk)], av)
        pltpu.sync_copy(c1_hbm.at[pl.ds(base, chunk)], bv)
        for i in range(chunk // _LANES):
            v0 = av[pl.ds(i * _LANES, _LANES)]
            v1 = bv[pl.ds(i * _LANES, _LANES)]
            t1 = jnp.bitwise_and(jnp.left_shift(v1, 9), _BUCKETS - 1)
            ov[pl.ds(i * _LANES, _LANES)] = jnp.bitwise_and(
                v0 + t1, _BUCKETS - 1)
        pltpu.sync_copy(ov, o_hbm.at[pl.ds(base, chunk)])


def kernel(tag, collect_aux, W_r, W_w, codebook_r, codebook_w):
    Bx, Tx, in_dim = tag.shape
    n = Bx * Tx
    x = tag.reshape(n, in_dim)
    cb2r = codebook_r.reshape(_HG, _K, _D) * 2.0
    cb2w = codebook_w.reshape(_HG, _K, _D) * 2.0
    aux = jnp.asarray(collect_aux, jnp.int32).reshape(1, 1)

    grid = (n // _TILE,)
    out_shape = (
        jax.ShapeDtypeStruct((n, _H), jnp.int32),
        jax.ShapeDtypeStruct((n, _H), jnp.int32),
        jax.ShapeDtypeStruct((n, _H), jnp.int32),
        jax.ShapeDtypeStruct((n, _H), jnp.int32),
        jax.ShapeDtypeStruct((n, _HG * _K), jnp.float32),
        jax.ShapeDtypeStruct((n, _HG * _K), jnp.float32),
    )
    in_specs = [
        pl.BlockSpec(memory_space=pltpu.SMEM),
        pl.BlockSpec((_TILE, in_dim), lambda i: (i, 0)),
        pl.BlockSpec((_ROUTE, in_dim), lambda i: (0, 0)),
        pl.BlockSpec((_ROUTE, in_dim), lambda i: (0, 0)),
        pl.BlockSpec((_HG, _K, _D), lambda i: (0, 0, 0)),
        pl.BlockSpec((_HG, _K, _D), lambda i: (0, 0, 0)),
    ]
    out_specs = (
        pl.BlockSpec((_TILE, _H), lambda i: (i, 0)),
        pl.BlockSpec((_TILE, _H), lambda i: (i, 0)),
        pl.BlockSpec((_TILE, _H), lambda i: (i, 0)),
        pl.BlockSpec((_TILE, _H), lambda i: (i, 0)),
        pl.BlockSpec((_TILE, _HG * _K), lambda i: (i, 0)),
        pl.BlockSpec((_TILE, _HG * _K), lambda i: (i, 0)),
    )
    c0r, c1r, c0w, c1w, lr, lw = pl.pallas_call(
        _body,
        grid=grid,
        in_specs=in_specs,
        out_specs=out_specs,
        out_shape=out_shape,
        scratch_shapes=[pltpu.VMEM((2 * _HG, _K), jnp.float32)],
    )(aux, x, W_r, W_w, cb2r, cb2w)

    mesh = plsc.VectorSubcoreMesh(core_axis_name="c", subcore_axis_name="s")
    sc = functools.partial(
        pl.kernel, mesh=mesh,
        out_type=(
            jax.ShapeDtypeStruct((2 * n,), jnp.int32),
            jax.ShapeDtypeStruct((2 * n,), jnp.int32),
        ),
        scratch_types=[
            pltpu.VMEM((2 * n // _NW,), jnp.int32),
            pltpu.VMEM((2 * n // _NW,), jnp.int32),
            pltpu.VMEM((2 * n // _NW,), jnp.int32),
        ],
    )(_sc_body)
    ir_flat, iw_flat = sc(c0r.reshape(-1), c1r.reshape(-1),
                          c0w.reshape(-1), c1w.reshape(-1))

    return (
        ir_flat.reshape(Bx, Tx, _H),
        iw_flat.reshape(Bx, Tx, _H),
        lr.reshape(Bx, Tx, _H, _G, _K),
        lw.reshape(Bx, Tx, _H, _G, _K),
    )
